# baseline (speedup 1.0000x reference)
import functools

import jax
import jax.numpy as jnp
from jax import lax
from jax.experimental import pallas as pl
from jax.experimental.pallas import tpu as pltpu
from jax.experimental.pallas import tpu_sc as plsc

N_CLS = 8
N_PRE = 5
N_SUF = 64
SEQ = 77
SEQ_PAD = 80
D = 512
CPB = 8


def _sc_counts():
    try:
        info = plsc.get_sparse_core_info()
        return int(info.num_cores), int(info.num_subcores)
    except Exception:
        return 2, 16


def _build_fused(token_prefix, cls_ctx, token_suffix):
    n_cls_total = cls_ctx.shape[0]

    def body(p_ref, c_ref, s_ref, o_ref):
        o_ref[:, 0:N_PRE] = jnp.broadcast_to(p_ref[...], (CPB, N_PRE, D))
        o_ref[:, N_PRE:N_PRE + N_CLS] = c_ref[...]
        o_ref[:, N_PRE + N_CLS:SEQ] = s_ref[...]
        o_ref[:, SEQ:SEQ_PAD] = jnp.zeros((CPB, SEQ_PAD - SEQ, D), jnp.float32)

    return pl.pallas_call(
        body,
        grid=(n_cls_total // CPB,),
        in_specs=[
            pl.BlockSpec((1, N_PRE, D), lambda i: (0, 0, 0)),
            pl.BlockSpec((CPB, N_CLS, D), lambda i: (i, 0, 0)),
            pl.BlockSpec((CPB, N_SUF, D), lambda i: (i, 0, 0)),
        ],
        out_specs=pl.BlockSpec((CPB, SEQ_PAD, D), lambda i: (i, 0, 0)),
        out_shape=jax.ShapeDtypeStruct((n_cls_total, SEQ_PAD, D), jnp.float32),
    )(token_prefix, cls_ctx, token_suffix)


def _gather_chunk(flat, idx_chunk, Bc, NC, NS):
    NW = NC * NS
    bw = Bc // NW

    mesh = plsc.VectorSubcoreMesh(core_axis_name="c", subcore_axis_name="s")

    @functools.partial(
        pl.kernel,
        mesh=mesh,
        out_type=jax.ShapeDtypeStruct((Bc, SEQ, D), jnp.float32),
        scratch_types=[
            pltpu.VMEM((bw * SEQ_PAD,), jnp.int32),
            pltpu.VMEM((2, SEQ_PAD, D), jnp.float32),
            pltpu.SemaphoreType.DMA,
            pltpu.SemaphoreType.DMA,
            pltpu.SemaphoreType.DMA,
            pltpu.SemaphoreType.DMA,
        ],
    )
    def _gather(flat_hbm, idx_hbm, out_hbm, idx_v, buf, g0, g1, w0, w1):
        wid = lax.axis_index("s") * NC + lax.axis_index("c")
        base = wid * bw
        pltpu.sync_copy(idx_hbm.at[pl.ds(base * SEQ_PAD, bw * SEQ_PAD)], idx_v)
        gsem = (g0, g1)
        wsem = (w0, w1)

        def fire_gather(j, h):
            pltpu.async_copy(
                flat_hbm.at[idx_v.at[pl.ds(j * SEQ_PAD, SEQ_PAD)]],
                buf.at[h], gsem[h])

        fire_gather(0, 0)
        fire_gather(1, 1)

        def body(k, carry):
            for h in (0, 1):
                j = 2 * k + h
                pltpu.make_async_copy(
                    flat_hbm.at[pl.ds(0, SEQ_PAD)], buf.at[h], gsem[h]).wait()
                pltpu.async_copy(
                    buf.at[h], out_hbm.at[base + j, pl.ds(0, SEQ_PAD)],
                    wsem[h])
            for h in (0, 1):
                pltpu.make_async_copy(
                    buf.at[h], out_hbm.at[0, pl.ds(0, SEQ_PAD)],
                    wsem[h]).wait()

                @pl.when(k < bw // 2 - 1)
                def _():
                    fire_gather(2 * k + h + 2, h)
            return carry

        lax.fori_loop(0, bw // 2, body, 0)

    return _gather(flat, idx_chunk)


def kernel(label, cls_ctx, token_prefix, token_suffix):
    B = label.shape[0]
    NC, NS = _sc_counts()

    fused = _build_fused(token_prefix, cls_ctx, token_suffix)
    flat = fused.reshape(fused.shape[0] * SEQ_PAD, D)
    lab = label.astype(jnp.int32)
    idx_all = (lab[:, None] * SEQ_PAD
               + jnp.arange(SEQ_PAD, dtype=jnp.int32)).reshape(-1)

    half = B // 2
    o1 = _gather_chunk(flat, idx_all[:half * SEQ_PAD], half, NC, NS)
    o2 = _gather_chunk(flat, idx_all[half * SEQ_PAD:], half, NC, NS)
    return jnp.concatenate([o1, o2], axis=0)


# speedup vs baseline: 2.3812x; 2.3812x over previous
import functools

import jax
import jax.numpy as jnp
from jax import lax
from jax.experimental import pallas as pl
from jax.experimental.pallas import tpu as pltpu
from jax.experimental.pallas import tpu_sc as plsc

N_CLS = 8
N_PRE = 5
N_SUF = 64
SEQ = 77
SEQ_PAD = 80
D = 512
CPB = 8
HB = 64


def _sc_counts():
    try:
        info = plsc.get_sparse_core_info()
        return int(info.num_cores), int(info.num_subcores)
    except Exception:
        return 2, 16


def _build_ftabt(token_prefix, cls_ctx, token_suffix):
    n_cls_total = cls_ctx.shape[0]

    def body(p_ref, c_ref, s_ref, o_ref):
        p = p_ref[0]
        o_ref[0:N_PRE] = jnp.broadcast_to(p[:, None, :], (N_PRE, CPB, D))
        o_ref[N_PRE:N_PRE + N_CLS] = jnp.swapaxes(c_ref[...], 0, 1)
        o_ref[N_PRE + N_CLS:SEQ] = jnp.swapaxes(s_ref[...], 0, 1)
        o_ref[SEQ:SEQ_PAD] = jnp.zeros((SEQ_PAD - SEQ, CPB, D), jnp.float32)

    return pl.pallas_call(
        body,
        grid=(n_cls_total // CPB,),
        in_specs=[
            pl.BlockSpec((1, N_PRE, D), lambda i: (0, 0, 0)),
            pl.BlockSpec((CPB, N_CLS, D), lambda i: (i, 0, 0)),
            pl.BlockSpec((CPB, N_SUF, D), lambda i: (i, 0, 0)),
        ],
        out_specs=pl.BlockSpec((SEQ_PAD, CPB, D), lambda i: (0, i, 0)),
        out_shape=jax.ShapeDtypeStruct((SEQ_PAD, n_cls_total, D), jnp.float32),
    )(token_prefix, cls_ctx, token_suffix)


def kernel(label, cls_ctx, token_prefix, token_suffix):
    B = label.shape[0]
    NC, NS = _sc_counts()
    NW = NC * NS
    bw = B // NW
    n_cls_total = cls_ctx.shape[0]

    ftabt = _build_ftabt(token_prefix, cls_ctx, token_suffix)
    flat = ftabt.reshape(SEQ_PAD * n_cls_total, D)
    lab = label.astype(jnp.int32)

    s_col = jnp.arange(SEQ, dtype=jnp.int32)[None, :, None]
    labw = lab.reshape(NW, 1, bw)
    idx_flat = (s_col * n_cls_total + labw).reshape(-1)

    mesh = plsc.VectorSubcoreMesh(core_axis_name="c", subcore_axis_name="s")

    @functools.partial(
        pl.kernel,
        mesh=mesh,
        out_type=jax.ShapeDtypeStruct((SEQ, B, D), jnp.float32),
        scratch_types=[
            pltpu.VMEM((SEQ * bw,), jnp.int32),
            pltpu.VMEM((2, HB, D), jnp.float32),
            pltpu.SemaphoreType.DMA,
            pltpu.SemaphoreType.DMA,
            pltpu.SemaphoreType.DMA,
            pltpu.SemaphoreType.DMA,
        ],
    )
    def _gather(flat_hbm, idx_hbm, out_hbm, idx_v, buf, g0, g1, w0, w1):
        wid = lax.axis_index("s") * NC + lax.axis_index("c")
        base = wid * bw
        pltpu.sync_copy(idx_hbm.at[pl.ds(base * SEQ, SEQ * bw)], idx_v)
        gsem = (g0, g1)
        wsem = (w0, w1)

        def fire_gather(s, h):
            pltpu.async_copy(
                flat_hbm.at[idx_v.at[pl.ds(s * bw + HB * h, HB)]],
                buf.at[h], gsem[h])

        fire_gather(0, 0)
        fire_gather(0, 1)

        def body(s, carry):
            for h in (0, 1):
                pltpu.make_async_copy(
                    flat_hbm.at[pl.ds(0, HB)], buf.at[h], gsem[h]).wait()
                pltpu.async_copy(
                    buf.at[h], out_hbm.at[s, pl.ds(base + HB * h, HB)],
                    wsem[h])
            for h in (0, 1):
                pltpu.make_async_copy(
                    buf.at[h], out_hbm.at[0, pl.ds(0, HB)], wsem[h]).wait()

                @pl.when(s < SEQ - 1)
                def _():
                    fire_gather(s + 1, h)
            return carry

        lax.fori_loop(0, SEQ, body, 0)

    res = _gather(flat, idx_flat)
    return jnp.transpose(res, (1, 0, 2))


# speedup vs baseline: 2.5677x; 1.0783x over previous
import functools

import jax
import jax.numpy as jnp
from jax import lax
from jax.experimental import pallas as pl
from jax.experimental.pallas import tpu as pltpu
from jax.experimental.pallas import tpu_sc as plsc

N_CLS = 8
N_PRE = 5
N_SUF = 64
SEQ = 77
SEQ_PAD = 80
D = 512
CPB = 40
HB = 64


def _sc_counts():
    try:
        info = plsc.get_sparse_core_info()
        return int(info.num_cores), int(info.num_subcores)
    except Exception:
        return 2, 16


def _build_ftabt(token_prefix, cls_ctx, token_suffix):
    n_cls_total = cls_ctx.shape[0]

    def body(p_ref, c_ref, s_ref, o_ref):
        p = p_ref[0]
        o_ref[0:N_PRE] = jnp.broadcast_to(p[:, None, :], (N_PRE, CPB, D))
        o_ref[N_PRE:N_PRE + N_CLS] = jnp.swapaxes(c_ref[...], 0, 1)
        o_ref[N_PRE + N_CLS:SEQ] = jnp.swapaxes(s_ref[...], 0, 1)
        o_ref[SEQ:SEQ_PAD] = jnp.zeros((SEQ_PAD - SEQ, CPB, D), jnp.float32)

    return pl.pallas_call(
        body,
        grid=(n_cls_total // CPB,),
        in_specs=[
            pl.BlockSpec((1, N_PRE, D), lambda i: (0, 0, 0)),
            pl.BlockSpec((CPB, N_CLS, D), lambda i: (i, 0, 0)),
            pl.BlockSpec((CPB, N_SUF, D), lambda i: (i, 0, 0)),
        ],
        out_specs=pl.BlockSpec((SEQ_PAD, CPB, D), lambda i: (0, i, 0)),
        out_shape=jax.ShapeDtypeStruct((SEQ_PAD, n_cls_total, D), jnp.float32),
    )(token_prefix, cls_ctx, token_suffix)


def kernel(label, cls_ctx, token_prefix, token_suffix):
    B = label.shape[0]
    NC, NS = _sc_counts()
    NW = NC * NS
    bw = B // NW
    n_cls_total = cls_ctx.shape[0]

    ftabt = _build_ftabt(token_prefix, cls_ctx, token_suffix)
    flat = ftabt.reshape(SEQ_PAD * n_cls_total, D)
    lab = label.astype(jnp.int32)

    s_col = jnp.arange(SEQ, dtype=jnp.int32)[None, :, None]
    labw = lab.reshape(NW, 1, bw)
    idx_flat = (s_col * n_cls_total + labw).reshape(-1)

    mesh = plsc.VectorSubcoreMesh(core_axis_name="c", subcore_axis_name="s")

    @functools.partial(
        pl.kernel,
        mesh=mesh,
        out_type=jax.ShapeDtypeStruct((SEQ, B, D), jnp.float32),
        scratch_types=[
            pltpu.VMEM((SEQ * bw,), jnp.int32),
            pltpu.VMEM((2, HB, D), jnp.float32),
            pltpu.SemaphoreType.DMA,
            pltpu.SemaphoreType.DMA,
            pltpu.SemaphoreType.DMA,
            pltpu.SemaphoreType.DMA,
        ],
    )
    def _gather(flat_hbm, idx_hbm, out_hbm, idx_v, buf, g0, g1, w0, w1):
        wid = lax.axis_index("s") * NC + lax.axis_index("c")
        base = wid * bw
        pltpu.sync_copy(idx_hbm.at[pl.ds(base * SEQ, SEQ * bw)], idx_v)
        gsem = (g0, g1)
        wsem = (w0, w1)

        def fire_gather(s, h):
            pltpu.async_copy(
                flat_hbm.at[idx_v.at[pl.ds(s * bw + HB * h, HB)]],
                buf.at[h], gsem[h])

        fire_gather(0, 0)
        fire_gather(0, 1)

        def body(s, carry):
            for h in (0, 1):
                pltpu.make_async_copy(
                    flat_hbm.at[pl.ds(0, HB)], buf.at[h], gsem[h]).wait()
                pltpu.async_copy(
                    buf.at[h], out_hbm.at[s, pl.ds(base + HB * h, HB)],
                    wsem[h])
            for h in (0, 1):
                pltpu.make_async_copy(
                    buf.at[h], out_hbm.at[0, pl.ds(0, HB)], wsem[h]).wait()

                @pl.when(s < SEQ - 1)
                def _():
                    fire_gather(s + 1, h)
            return carry

        lax.fori_loop(0, SEQ, body, 0)

    res = _gather(flat, idx_flat)
    return jnp.transpose(res, (1, 0, 2))


# speedup vs baseline: 2.6540x; 1.0336x over previous
import functools

import jax
import jax.numpy as jnp
from jax import lax
from jax.experimental import pallas as pl
from jax.experimental.pallas import tpu as pltpu
from jax.experimental.pallas import tpu_sc as plsc

N_CLS = 8
N_PRE = 5
N_SUF = 64
SEQ = 77
SEQ_PAD = 80
D = 512
CPB = 40
HB = 64


def _sc_counts():
    try:
        info = plsc.get_sparse_core_info()
        return int(info.num_cores), int(info.num_subcores)
    except Exception:
        return 2, 16


def _build_ftabt(token_prefix, cls_ctx, token_suffix):
    n_cls_total = cls_ctx.shape[0]

    def body(p_ref, c_ref, s_ref, o_ref):
        p = p_ref[0]
        o_ref[0:N_PRE] = jnp.broadcast_to(p[:, None, :], (N_PRE, CPB, D))
        o_ref[N_PRE:N_PRE + N_CLS] = jnp.swapaxes(c_ref[...], 0, 1)
        o_ref[N_PRE + N_CLS:SEQ] = jnp.swapaxes(s_ref[...], 0, 1)
        o_ref[SEQ:SEQ_PAD] = jnp.zeros((SEQ_PAD - SEQ, CPB, D), jnp.float32)

    return pl.pallas_call(
        body,
        grid=(n_cls_total // CPB,),
        in_specs=[
            pl.BlockSpec((1, N_PRE, D), lambda i: (0, 0, 0)),
            pl.BlockSpec((CPB, N_CLS, D), lambda i: (i, 0, 0)),
            pl.BlockSpec((CPB, N_SUF, D), lambda i: (i, 0, 0)),
        ],
        out_specs=pl.BlockSpec((SEQ_PAD, CPB, D), lambda i: (0, i, 0)),
        out_shape=jax.ShapeDtypeStruct((SEQ_PAD, n_cls_total, D), jnp.float32),
    )(token_prefix, cls_ctx, token_suffix)


def kernel(label, cls_ctx, token_prefix, token_suffix):
    B = label.shape[0]
    NC, NS = _sc_counts()
    NW = NC * NS
    bw = B // NW
    n_cls_total = cls_ctx.shape[0]

    ftabt = _build_ftabt(token_prefix, cls_ctx, token_suffix)
    flat = ftabt.reshape(SEQ_PAD * n_cls_total, D)
    lab = label.astype(jnp.int32)

    s_col = jnp.arange(SEQ, dtype=jnp.int32)[None, :, None]
    labw = lab.reshape(NW, 1, bw)
    idx_flat = (s_col * n_cls_total + labw).reshape(-1)

    mesh = plsc.VectorSubcoreMesh(core_axis_name="c", subcore_axis_name="s")

    @functools.partial(
        pl.kernel,
        mesh=mesh,
        out_type=jax.ShapeDtypeStruct((SEQ, B, D), jnp.float32),
        scratch_types=[
            pltpu.VMEM((SEQ * bw,), jnp.int32),
            pltpu.VMEM((3, HB, D), jnp.float32),
            pltpu.SemaphoreType.DMA,
            pltpu.SemaphoreType.DMA,
            pltpu.SemaphoreType.DMA,
            pltpu.SemaphoreType.DMA,
            pltpu.SemaphoreType.DMA,
            pltpu.SemaphoreType.DMA,
        ],
    )
    def _gather(flat_hbm, idx_hbm, out_hbm, idx_v, buf,
                g0, g1, g2, w0, w1, w2):
        wid = lax.axis_index("s") * NC + lax.axis_index("c")
        base = wid * bw
        pltpu.sync_copy(idx_hbm.at[pl.ds(base * SEQ, SEQ * bw)], idx_v)
        gsem = (g0, g1, g2)
        wsem = (w0, w1, w2)
        NSTEP = 2 * SEQ

        def fire_gather(s, h, st):
            pltpu.async_copy(
                flat_hbm.at[idx_v.at[pl.ds(s * bw + HB * h, HB)]],
                buf.at[st], gsem[st])

        def step(s, h, st):
            pltpu.make_async_copy(
                flat_hbm.at[pl.ds(0, HB)], buf.at[st], gsem[st]).wait()
            pltpu.async_copy(
                buf.at[st], out_hbm.at[s, pl.ds(base + HB * h, HB)],
                wsem[st])
            pltpu.make_async_copy(
                buf.at[st], out_hbm.at[0, pl.ds(0, HB)], wsem[st]).wait()

        for t in range(3):
            fire_gather(t // 2, t % 2, t % 3)

        def body(k, carry):
            for i in range(6):
                t_s = 3 * k + (i // 2)
                step(t_s, i % 2, i % 3)
                nxt = i + 3
                fire_gather(3 * k + nxt // 2, nxt % 2, i % 3)
            return carry

        lax.fori_loop(0, (NSTEP - 4) // 6, body, 0)
        for t in range(NSTEP - 4, NSTEP):
            step(t // 2, t % 2, t % 3)
            if t + 3 < NSTEP:
                fire_gather((t + 3) // 2, (t + 3) % 2, t % 3)

    res = _gather(flat, idx_flat)
    return jnp.transpose(res, (1, 0, 2))
